# Initial kernel scaffold; baseline (speedup 1.0000x reference)
import jax
import jax.numpy as jnp
from jax.experimental import pallas as pl


def kernel(x, edge_index, edge_attr, W1, root1, bias1, W2, root2, bias2):
    raise NotImplementedError("write your pallas kernel here")


# speedup vs baseline: 2.6806x; 2.6806x over previous
import functools

import jax
import jax.numpy as jnp
from jax import lax
from jax.experimental import pallas as pl
from jax.experimental.pallas import tpu as pltpu
from jax.experimental.pallas import tpu_sc as plsc

_NC = 2
_NS = 16
_L = 16
_CHUNK = 80


def _pad_rows(N):
    rpt = (-(-N // _NS) + 7) // 8 * 8
    return rpt, rpt * _NS


@functools.lru_cache(maxsize=None)
def _make_edge_accum(N, E, F, with_cnt, interpret=False):
    assert E % (_NS * _CHUNK) == 0, (E, _NS, _CHUNK)
    assert F % _L == 0
    edges_per_tile = E // _NS
    nchunk = edges_per_tile // _CHUNK
    rpt, NP = _pad_rows(N)
    nj = F // _L

    out_type = [jax.ShapeDtypeStruct((_NC * NP, F), jnp.float32)]
    scratch = [
        pltpu.VMEM_SHARED((NP, F), jnp.float32),
        pltpu.VMEM((_CHUNK,), jnp.int32),
        pltpu.VMEM((_CHUNK,), jnp.int32),
        pltpu.VMEM((_CHUNK,), jnp.float32),
        pltpu.VMEM((_CHUNK, F), jnp.float32),
        pltpu.VMEM((8, F), jnp.float32),
        pltpu.SemaphoreType.DMA,
    ]
    if with_cnt:
        out_type.append(jax.ShapeDtypeStruct((_NS * NP,), jnp.float32))
        scratch.append(pltpu.VMEM((NP,), jnp.float32))

    def body(table, src_h, dst_h, u_h, *rest):
        if with_cnt:
            (sa_out, cnt_out, acc_sp, idx_v, dst_v, u_v, rows_v, zrow_v,
             sem, cnt_v) = rest
        else:
            (sa_out, acc_sp, idx_v, dst_v, u_v, rows_v, zrow_v, sem) = rest

        c = lax.axis_index("c")
        s = lax.axis_index("s")
        zeros16 = jnp.zeros((_L,), jnp.float32)
        ones16 = jnp.ones((_L,), jnp.float32)

        def zbody(i, _):
            for j in range(nj):
                zrow_v[i, pl.ds(_L * j, _L)] = zeros16
            return 0
        lax.fori_loop(0, 8, zbody, 0)
        if with_cnt:
            def cbody(i, _):
                cnt_v[pl.ds(i * _L, _L)] = zeros16
                return 0
            lax.fori_loop(0, NP // _L, cbody, 0)
        rbase = s * rpt

        def zcopy(i, _):
            pltpu.sync_copy(zrow_v, acc_sp.at[pl.ds(rbase + i * 8, 8)])
            return 0
        lax.fori_loop(0, rpt // 8, zcopy, 0)
        plsc.subcore_barrier()

        ebase = s * edges_per_tile

        def chunk(g, _):
            eb = ebase + g * _CHUNK
            pltpu.sync_copy(src_h.at[pl.ds(eb, _CHUNK)], idx_v)
            pltpu.sync_copy(dst_h.at[pl.ds(eb, _CHUNK)], dst_v)
            pltpu.sync_copy(u_h.at[pl.ds(eb, _CHUNK)], u_v)
            pltpu.async_copy(table.at[idx_v], rows_v, sem).wait()

            @pl.when(c == 1)
            def _():
                def mul(q, _):
                    ub = u_v[pl.ds(q * _L, _L)]
                    for t in range(_L):
                        e = q * _L + t
                        us = ub[t]
                        for j in range(nj):
                            sl = pl.ds(_L * j, _L)
                            rows_v[e, sl] = rows_v[e, sl] * us
                    return 0
                lax.fori_loop(0, _CHUNK // _L, mul, 0)

            pltpu.sync_copy(rows_v, acc_sp.at[dst_v], add=True)

            if with_cnt:
                @pl.when(c == 0)
                def _():
                    for q in range(_CHUNK // _L):
                        dst16 = dst_v[pl.ds(q * _L, _L)]
                        plsc.addupdate_scatter(cnt_v, [dst16], ones16)
            return 0

        lax.fori_loop(0, nchunk, chunk, 0)
        plsc.subcore_barrier()

        obase = c * NP + rbase
        pltpu.sync_copy(acc_sp.at[pl.ds(rbase, rpt)],
                        sa_out.at[pl.ds(obase, rpt)])
        if with_cnt:
            @pl.when(c == 0)
            def _():
                pltpu.sync_copy(cnt_v, cnt_out.at[pl.ds(s * NP, NP)])

    mesh = plsc.VectorSubcoreMesh(core_axis_name="c", subcore_axis_name="s",
                                  num_cores=_NC, num_subcores=_NS)
    out_arg = tuple(out_type) if with_cnt else out_type[0]
    return pl.kernel(body, out_type=out_arg, mesh=mesh,
                     scratch_types=tuple(scratch), interpret=interpret,
                     compiler_params=pltpu.CompilerParams(
                         needs_layout_passes=False))


@functools.lru_cache(maxsize=None)
def _make_dense_layer(N, Fin, Fout, apply_elu, interpret=False):
    BN = 1000 if N % 1000 == 0 else N
    assert N % BN == 0
    grid = (N // BN,)

    def dot(a, b):
        return lax.dot_general(a, b, (((1,), (0,)), ((), ())),
                               preferred_element_type=jnp.float32)

    def body(s_ref, a_ref, cnt_ref, x_ref, w_ref, root_ref, bias_ref, o_ref):
        sm = s_ref[0]
        am = a_ref[0]
        agg = dot(sm - am, w_ref[0]) + dot(am, w_ref[1])
        cnt = jnp.sum(cnt_ref[...], axis=1, keepdims=True)
        agg = agg / jnp.maximum(cnt, 1.0)
        r = agg + dot(x_ref[...], root_ref[...]) + bias_ref[...]
        if apply_elu:
            r = jnp.where(r > 0, r, jnp.exp(jnp.minimum(r, 0.0)) - 1.0)
        o_ref[...] = r

    return pl.pallas_call(
        body,
        grid=grid,
        in_specs=[
            pl.BlockSpec((1, BN, Fin), lambda i: (0, i, 0)),
            pl.BlockSpec((1, BN, Fin), lambda i: (1, i, 0)),
            pl.BlockSpec((BN, _NS), lambda i: (i, 0)),
            pl.BlockSpec((BN, Fin), lambda i: (i, 0)),
            pl.BlockSpec((2, Fin, Fout), lambda i: (0, 0, 0)),
            pl.BlockSpec((Fin, Fout), lambda i: (0, 0)),
            pl.BlockSpec((1, Fout), lambda i: (0, 0)),
        ],
        out_specs=pl.BlockSpec((BN, Fout), lambda i: (i, 0)),
        out_shape=jax.ShapeDtypeStruct((N, Fout), jnp.float32),
        interpret=interpret,
    )


def kernel(x, edge_index, edge_attr, W1, root1, bias1, W2, root2, bias2):
    N, F = x.shape
    E = edge_index.shape[1]
    H = root1.shape[1]
    C = root2.shape[1]
    src = edge_index[0]
    dst = edge_index[1]
    u = edge_attr[:, 0]
    _, NP = _pad_rows(N)

    SA1, cntf = _make_edge_accum(N, E, F, True)(x, src, dst, u)
    SA1 = SA1.reshape(_NC, NP, F)
    cnt16 = cntf.reshape(_NS, NP).T
    h = _make_dense_layer(N, F, H, True)(
        SA1, SA1, cnt16, x, W1, root1, bias1.reshape(1, H))

    SA2 = _make_edge_accum(N, E, H, False)(h, src, dst, u)
    SA2 = SA2.reshape(_NC, NP, H)
    out = _make_dense_layer(N, H, C, False)(
        SA2, SA2, cnt16, h, W2, root2, bias2.reshape(1, C))
    return out
